# baseline (speedup 1.0000x reference)
import jax
import jax.numpy as jnp
from jax import lax
from jax.experimental import pallas as pl
from jax.experimental.pallas import tpu as pltpu
from jax.experimental.pallas import tpu_sc as plsc

N_FIELDS = 26
VOCAB = 100000
DIM = 32
B = 16384
NC, NS, L = 2, 16, 16
NW = NC * NS
BPW = B // NW
NTC = VOCAB // 128
QROWS = VOCAB // 4

_CP = pltpu.CompilerParams(use_tc_tiling_on_sc=True,
                           needs_layout_passes=False)


def _repack_body(Wt_hbm, Wq_hbm, src0, src1, wq0, wq1, isem, osem):
    wid = lax.axis_index("s") * NC + lax.axis_index("c")
    lane = lax.broadcasted_iota(jnp.int32, (L,), 0)
    src = [src0, src1]
    wq = [wq0, wq1]

    def in_copy(j, c, b):
        return pltpu.make_async_copy(Wt_hbm.at[j, :, pl.ds(c * 128, 128)],
                                     src[b], isem.at[b])

    def out_copy(j, c, b):
        return pltpu.make_async_copy(
            wq[b], Wq_hbm.at[j, pl.ds(c * 32, 32), :], osem.at[b])

    def transpose_block(b, nrows):
        for m in range(nrows):
            for q in range(4):
                cvec = lane * 0 + (4 * m + q)
                for dh in range(2):
                    g = plsc.load_gather(src[b], [lane + dh * L, cvec])
                    wq[b][m, pl.ds(q * 32 + dh * L, L)] = g

    def _i(i, carry):
        c = i * 32 + wid
        in_copy(0, c, 0).start()
        in_copy(1, c, 1).start()

        def _j2(j2, carry2):
            for b in range(2):
                j = j2 * 2 + b
                in_copy(j, c, b).wait()
                transpose_block(b, 32)
                out_copy(j, c, b).start()
                nxt = j + 2

                @pl.when(nxt < N_FIELDS)
                def _():
                    in_copy(nxt, c, b).start()

                @pl.when(j >= 2)
                def _():
                    out_copy(j - 2, c, b).wait()
            return carry2

        lax.fori_loop(0, N_FIELDS // 2, _j2, 0)
        out_copy(N_FIELDS - 2, c, 0).wait()
        out_copy(N_FIELDS - 1, c, 1).wait()
        return carry

    lax.fori_loop(0, 24, _i, 0)

    if True:
        c = 24 * 32 + wid

        @pl.when(c < NTC)
        def _():
            def _jm(j, carry):
                in_copy(j, c, 0).start()
                in_copy(j, c, 0).wait()
                transpose_block(0, 32)
                out_copy(j, c, 0).start()
                out_copy(j, c, 0).wait()
                return carry

            lax.fori_loop(0, N_FIELDS, _jm, 0)

            @pl.when(c == NTC)
            def _():
                def _jt(j, carry):
                    cps = [pltpu.make_async_copy(
                        Wt_hbm.at[j, d, pl.ds(NTC * 128, 32)],
                        src0.at[d, pl.ds(0, 32)], isem.at[0])
                        for d in range(DIM)]
                    for cp in cps:
                        cp.start()
                    for cp in cps:
                        cp.wait()
                    transpose_block(0, 8)
                    tout = pltpu.make_async_copy(
                        wq0.at[pl.ds(0, 8), :],
                        Wq_hbm.at[j, pl.ds(NTC * 32, 8), :], osem.at[0])
                    tout.start()
                    tout.wait()
                    return carry

                lax.fori_loop(0, N_FIELDS, _jt, 0)


def _gather_body(xT_hbm, Wq_hbm, out_hbm, idxb, idx4, qcol, rows2, obuf,
                 gsem, osem):
    wid = lax.axis_index("s") * NC + lax.axis_index("c")
    base = wid * BPW
    lane = lax.broadcasted_iota(jnp.int32, (L,), 0)

    def _field(j, carry):
        pltpu.sync_copy(xT_hbm.at[j, pl.ds(base, BPW)], idxb)

        def _prep(i, c2):
            g = idxb[pl.ds(i * L, L)]
            idx4[pl.ds(i * L, L)] = jax.lax.shift_right_logical(g, 2)
            qcol[pl.ds(i * L, L)] = jax.lax.shift_left(
                jax.lax.bitwise_and(g, 3), 5)
            return c2

        lax.fori_loop(0, BPW // L, _prep, 0)
        pltpu.async_copy(Wq_hbm.at[j].at[idx4], rows2, gsem).wait()

        for bg in range(BPW // L):
            rv = lane + bg * L
            cq = qcol[pl.ds(bg * L, L)]
            for d in range(DIM):
                g = plsc.load_gather(rows2, [rv, cq + d])
                obuf[d, pl.ds(bg * L, L)] = g

        pltpu.async_copy(obuf, out_hbm.at[j, :, pl.ds(base, BPW)],
                         osem).wait()
        return carry

    lax.fori_loop(0, N_FIELDS, _field, 0)


def kernel(x, W):
    xT = x.T
    Wt = jnp.swapaxes(W, 1, 2)
    mesh = plsc.VectorSubcoreMesh(
        core_axis_name="c", subcore_axis_name="s",
        num_cores=NC, num_subcores=NS,
    )
    Wq = pl.kernel(
        _repack_body,
        out_type=jax.ShapeDtypeStruct((N_FIELDS, QROWS, 128), jnp.float32),
        mesh=mesh,
        scratch_types=[
            pltpu.VMEM((DIM, 128), jnp.float32),
            pltpu.VMEM((DIM, 128), jnp.float32),
            pltpu.VMEM((DIM, 128), jnp.float32),
            pltpu.VMEM((DIM, 128), jnp.float32),
            pltpu.SemaphoreType.DMA((2,)),
            pltpu.SemaphoreType.DMA((2,)),
        ],
        compiler_params=_CP,
    )(Wt)
    out = pl.kernel(
        _gather_body,
        out_type=jax.ShapeDtypeStruct((N_FIELDS, DIM, B), jnp.float32),
        mesh=mesh,
        scratch_types=[
            pltpu.VMEM((BPW,), jnp.int32),
            pltpu.VMEM((BPW,), jnp.int32),
            pltpu.VMEM((BPW,), jnp.int32),
            pltpu.VMEM((BPW, 128), jnp.float32),
            pltpu.VMEM((DIM, BPW), jnp.float32),
            pltpu.SemaphoreType.DMA,
            pltpu.SemaphoreType.DMA,
        ],
        compiler_params=_CP,
    )(xT, Wq)
    return jnp.swapaxes(out, 1, 2)


# speedup vs baseline: 1.1245x; 1.1245x over previous
import jax
import jax.numpy as jnp
from jax import lax
from jax.experimental import pallas as pl
from jax.experimental.pallas import tpu as pltpu
from jax.experimental.pallas import tpu_sc as plsc

N_FIELDS = 26
VOCAB = 100000
DIM = 32
B = 16384
NC, NS, L = 2, 16, 16
NW = NC * NS
BPW = B // NW
NTC = VOCAB // 128
QROWS = VOCAB // 4

_CP = pltpu.CompilerParams(use_tc_tiling_on_sc=True,
                           needs_layout_passes=False)


def _repack_body(Wt_hbm, Wq_hbm, src0, src1, wq0, wq1, isem, osem):
    wid = lax.axis_index("s") * NC + lax.axis_index("c")
    lane = lax.broadcasted_iota(jnp.int32, (L,), 0)
    src = [src0, src1]
    wq = [wq0, wq1]

    def in_copy(j, c, b):
        return pltpu.make_async_copy(Wt_hbm.at[j, :, pl.ds(c * 128, 128)],
                                     src[b], isem.at[b])

    def out_copy(j, c, b):
        return pltpu.make_async_copy(
            wq[b], Wq_hbm.at[j, pl.ds(c * 32, 32), :], osem.at[b])

    def transpose_block(b, nrows):
        def _m4(m4, carry):
            base16 = lane * 0 + m4 * 16
            for mo in range(4):
                m = m4 * 4 + mo
                for q in range(4):
                    cvec = base16 + (4 * mo + q)
                    for dh in range(2):
                        g = plsc.load_gather(src[b], [lane + dh * L, cvec])
                        wq[b][m, pl.ds(q * 32 + dh * L, L)] = g
            return carry

        lax.fori_loop(0, nrows // 4, _m4, 0)

    def _i(i, carry):
        c = i * 32 + wid
        in_copy(0, c, 0).start()
        in_copy(1, c, 1).start()

        def _j2(j2, carry2):
            for b in range(2):
                j = j2 * 2 + b
                in_copy(j, c, b).wait()
                transpose_block(b, 32)
                out_copy(j, c, b).start()
                nxt = j + 2

                @pl.when(nxt < N_FIELDS)
                def _():
                    in_copy(nxt, c, b).start()

                @pl.when(j >= 2)
                def _():
                    out_copy(j - 2, c, b).wait()
            return carry2

        lax.fori_loop(0, N_FIELDS // 2, _j2, 0)
        out_copy(N_FIELDS - 2, c, 0).wait()
        out_copy(N_FIELDS - 1, c, 1).wait()
        return carry

    lax.fori_loop(0, 24, _i, 0)

    if True:
        c = 24 * 32 + wid

        @pl.when(c < NTC)
        def _():
            def _jm(j, carry):
                in_copy(j, c, 0).start()
                in_copy(j, c, 0).wait()
                transpose_block(0, 32)
                out_copy(j, c, 0).start()
                out_copy(j, c, 0).wait()
                return carry

            lax.fori_loop(0, N_FIELDS, _jm, 0)

            @pl.when(c == NTC)
            def _():
                def _jt(j, carry):
                    cps = [pltpu.make_async_copy(
                        Wt_hbm.at[j, d, pl.ds(NTC * 128, 32)],
                        src0.at[d, pl.ds(0, 32)], isem.at[0])
                        for d in range(DIM)]
                    for cp in cps:
                        cp.start()
                    for cp in cps:
                        cp.wait()
                    transpose_block(0, 8)
                    tout = pltpu.make_async_copy(
                        wq0.at[pl.ds(0, 8), :],
                        Wq_hbm.at[j, pl.ds(NTC * 32, 8), :], osem.at[0])
                    tout.start()
                    tout.wait()
                    return carry

                lax.fori_loop(0, N_FIELDS, _jt, 0)


def _gather_body(xT_hbm, Wq_hbm, out_hbm, idxb, idx4, qcol, rows2,
                 *rest):
    pieces = [list(rest[4 * r:4 * r + 4]) for r in range(4)]
    gsem, osem = rest[16], rest[17]
    wid = lax.axis_index("s") * NC + lax.axis_index("c")
    base = wid * BPW
    lane = lax.broadcasted_iota(jnp.int32, (L,), 0)

    def _field(j, carry):
        pltpu.sync_copy(xT_hbm.at[j, pl.ds(base, BPW)], idxb)

        def _prep(i, c2):
            g = idxb[pl.ds(i * L, L)]
            idx4[pl.ds(i * L, L)] = jax.lax.shift_right_logical(g, 2)
            qcol[pl.ds(i * L, L)] = jax.lax.shift_left(
                jax.lax.bitwise_and(g, 3), 5)
            return c2

        lax.fori_loop(0, BPW // L, _prep, 0)
        pltpu.async_copy(Wq_hbm.at[j].at[idx4], rows2, gsem).wait()

        for cb in range(4):
            def _bgw(bgw, c2):
                boff = cb * 8 * L + bgw * L
                rv = lane + boff
                cq = qcol[pl.ds(boff, L)]
                for d in range(DIM):
                    g = plsc.load_gather(rows2, [rv, cq + d])
                    pieces[d // 8][cb][d % 8, pl.ds(bgw * L, L)] = g
                return c2

            lax.fori_loop(0, 8, _bgw, 0)
        for r in range(4):
            for cb in range(4):
                pltpu.sync_copy(
                    pieces[r][cb],
                    out_hbm.at[j, pl.ds(8 * r, 8),
                               pl.ds(base + 128 * cb, 128)])
        return carry

    lax.fori_loop(0, N_FIELDS, _field, 0)


def kernel(x, W):
    xT = x.T
    Wt = jnp.swapaxes(W, 1, 2)
    mesh = plsc.VectorSubcoreMesh(
        core_axis_name="c", subcore_axis_name="s",
        num_cores=NC, num_subcores=NS,
    )
    Wq = pl.kernel(
        _repack_body,
        out_type=jax.ShapeDtypeStruct((N_FIELDS, QROWS, 128), jnp.float32),
        mesh=mesh,
        scratch_types=[
            pltpu.VMEM((DIM, 128), jnp.float32),
            pltpu.VMEM((DIM, 128), jnp.float32),
            pltpu.VMEM((DIM, 128), jnp.float32),
            pltpu.VMEM((DIM, 128), jnp.float32),
            pltpu.SemaphoreType.DMA((2,)),
            pltpu.SemaphoreType.DMA((2,)),
        ],
        compiler_params=_CP,
    )(Wt)
    out = pl.kernel(
        _gather_body,
        out_type=jax.ShapeDtypeStruct((N_FIELDS, DIM, B), jnp.float32),
        mesh=mesh,
        scratch_types=[
            pltpu.VMEM((BPW,), jnp.int32),
            pltpu.VMEM((BPW,), jnp.int32),
            pltpu.VMEM((BPW,), jnp.int32),
            pltpu.VMEM((BPW, 128), jnp.float32),
        ] + [pltpu.VMEM((8, 128), jnp.float32) for _ in range(16)] + [
            pltpu.SemaphoreType.DMA,
            pltpu.SemaphoreType.DMA,
        ],
        compiler_params=_CP,
    )(xT, Wq)
    return jnp.swapaxes(out, 1, 2)


# speedup vs baseline: 2.0225x; 1.7986x over previous
import jax
import jax.numpy as jnp
from jax import lax
from jax.experimental import pallas as pl
from jax.experimental.pallas import tpu as pltpu
from jax.experimental.pallas import tpu_sc as plsc

N_FIELDS = 26
VOCAB = 100000
DIM = 32
B = 16384
NC, NS, L = 2, 16, 16
NW = NC * NS
BPW = B // NW
NBUF = 3


def _gather_body(x_hbm, W_hbm, out_hbm, xblk_v, idx_v, rows_v, gsem, osem):
    wid = lax.axis_index("s") * NC + lax.axis_index("c")
    base = wid * BPW

    pltpu.sync_copy(x_hbm.at[pl.ds(base * N_FIELDS, BPW * N_FIELDS)], xblk_v)
    lane = lax.broadcasted_iota(jnp.int32, (L,), 0)

    def load_idx(j, s):
        col = lane * N_FIELDS + j

        def _mk(i, carry):
            g = plsc.load_gather(xblk_v, [col + i * (L * N_FIELDS)])
            idx_v[s, pl.ds(i * L, L)] = g
            return carry

        lax.fori_loop(0, BPW // L, _mk, 0)

    def gather(j, s):
        return pltpu.make_async_copy(W_hbm.at[j].at[idx_v.at[s]],
                                     rows_v.at[s], gsem.at[s])

    def writeback(j, s):
        return pltpu.make_async_copy(rows_v.at[s],
                                     out_hbm.at[j, pl.ds(base, BPW)],
                                     osem.at[s])

    for j in range(N_FIELDS + 1):
        if j < N_FIELDS:
            s = j % NBUF
            if j >= NBUF:
                writeback(j - NBUF, s).wait()
            load_idx(j, s)
            gather(j, s).start()
        if 1 <= j:
            s1 = (j - 1) % NBUF
            gather(j - 1, s1).wait()
            writeback(j - 1, s1).start()
    for j in range(N_FIELDS - NBUF + 1, N_FIELDS):
        writeback(j, j % NBUF).wait()


def kernel(x, W):
    mesh = plsc.VectorSubcoreMesh(
        core_axis_name="c", subcore_axis_name="s",
        num_cores=NC, num_subcores=NS,
    )
    return pl.kernel(
        _gather_body,
        out_type=jax.ShapeDtypeStruct((N_FIELDS, B, DIM), jnp.float32),
        mesh=mesh,
        scratch_types=[
            pltpu.VMEM((BPW * N_FIELDS,), jnp.int32),
            pltpu.VMEM((NBUF, BPW), jnp.int32),
            pltpu.VMEM((NBUF, BPW, DIM), jnp.float32),
            pltpu.SemaphoreType.DMA((NBUF,)),
            pltpu.SemaphoreType.DMA((NBUF,)),
        ],
        compiler_params=pltpu.CompilerParams(use_tc_tiling_on_sc=False,
                                             needs_layout_passes=False),
    )(x.reshape(B * N_FIELDS), W)
